# baseline (speedup 1.0000x reference)
import functools

import jax
import jax.numpy as jnp
from jax import lax
from jax.experimental import pallas as pl
from jax.experimental.pallas import tpu as pltpu
from jax.experimental.pallas import tpu_sc as plsc

_NC = 2
_NS = 16
_L = 16


def _sc_bprmf(B, K, n_users, n_items):
    NW = _NC * _NS
    n = B // NW
    CH = 256
    HH = CH // 2
    NCH = n // CH
    NG = HH // _L

    mesh = plsc.VectorSubcoreMesh(core_axis_name="c", subcore_axis_name="s")

    @functools.partial(
        pl.kernel,
        mesh=mesh,
        out_type=(
            jax.ShapeDtypeStruct((B,), jnp.float32),
            jax.ShapeDtypeStruct((B,), jnp.float32),
        ),
        scratch_types=[
            pltpu.VMEM((n,), jnp.int32),
            pltpu.VMEM((n,), jnp.int32),
            pltpu.VMEM((n,), jnp.int32),
            pltpu.VMEM((HH, K), jnp.float32),
            pltpu.VMEM((HH, K), jnp.float32),
            pltpu.VMEM((HH, K), jnp.float32),
            pltpu.VMEM((HH, K), jnp.float32),
            pltpu.VMEM((HH, K), jnp.float32),
            pltpu.VMEM((HH, K), jnp.float32),
            pltpu.VMEM((n,), jnp.float32),
            pltpu.VMEM((n,), jnp.float32),
            pltpu.SemaphoreType.DMA,
        ],
        compiler_params=pltpu.CompilerParams(needs_layout_passes=False),
    )
    def sc_kernel(u_hbm, ip_hbm, in_hbm, ue_hbm, ie_hbm, sp_hbm, sn_hbm,
                  u_idx, ip_idx, in_idx, ua, ub, pa, pb, na, nb,
                  sp_v, sn_v, sem):
        wid = lax.axis_index("s") * _NC + lax.axis_index("c")
        base = wid * n

        off = pl.ds(base, n)
        pltpu.sync_copy(u_hbm.at[off], u_idx)
        pltpu.sync_copy(ip_hbm.at[off], ip_idx)
        pltpu.sync_copy(in_hbm.at[off], in_idx)

        lanes = lax.iota(jnp.int32, _L)

        def chunk_body(c, carry):
            c0 = pl.multiple_of(c * CH, CH)

            def fire_body(g, carry2):
                r0 = pl.multiple_of(g * _L, _L)
                uv0 = u_idx[pl.ds(c0 + r0, _L)]
                pv0 = ip_idx[pl.ds(c0 + r0, _L)]
                nv0 = in_idx[pl.ds(c0 + r0, _L)]
                uv1 = u_idx[pl.ds(c0 + HH + r0, _L)]
                pv1 = ip_idx[pl.ds(c0 + HH + r0, _L)]
                nv1 = in_idx[pl.ds(c0 + HH + r0, _L)]
                for t in range(_L):
                    r = pl.ds(r0 + t, 1)
                    pltpu.async_copy(ue_hbm.at[pl.ds(uv0[t], 1)], ua.at[r], sem)
                    pltpu.async_copy(ie_hbm.at[pl.ds(pv0[t], 1)], pa.at[r], sem)
                    pltpu.async_copy(ie_hbm.at[pl.ds(nv0[t], 1)], na.at[r], sem)
                    pltpu.async_copy(ue_hbm.at[pl.ds(uv1[t], 1)], ub.at[r], sem)
                    pltpu.async_copy(ie_hbm.at[pl.ds(pv1[t], 1)], pb.at[r], sem)
                    pltpu.async_copy(ie_hbm.at[pl.ds(nv1[t], 1)], nb.at[r], sem)
                return carry2

            lax.fori_loop(0, NG, fire_body, 0)

            for buf in (ua, ub, pa, pb, na, nb):
                pltpu.make_async_copy(ue_hbm.at[pl.ds(0, HH)], buf, sem).wait()

            def make_group_body(uu, pp, nn, half_off):
                def group_body(g, carry2):
                    row0 = pl.multiple_of(g * _L, _L)
                    rows = row0 + lanes
                    acc_p = jnp.zeros((_L,), jnp.float32)
                    acc_n = jnp.zeros((_L,), jnp.float32)
                    for k in range(K):
                        col = jnp.full((_L,), k, jnp.int32)
                        ue_k = plsc.load_gather(uu, [rows, col])
                        ip_k = plsc.load_gather(pp, [rows, col])
                        in_k = plsc.load_gather(nn, [rows, col])
                        acc_p = acc_p + ue_k * ip_k
                        acc_n = acc_n + ue_k * in_k
                    sp_v[pl.ds(c0 + half_off + row0, _L)] = acc_p
                    sn_v[pl.ds(c0 + half_off + row0, _L)] = acc_n
                    return carry2
                return group_body

            lax.fori_loop(0, NG, make_group_body(ua, pa, na, 0), 0)
            lax.fori_loop(0, NG, make_group_body(ub, pb, nb, HH), 0)
            return carry

        lax.fori_loop(0, NCH, chunk_body, 0)

        out_off = pl.ds(base, n)
        pltpu.sync_copy(sp_v, sp_hbm.at[out_off])
        pltpu.sync_copy(sn_v, sn_hbm.at[out_off])

    return sc_kernel


def kernel(u, i_pos, i_neg, user_emb, item_emb):
    B = u.shape[0]
    n_users, K = user_emb.shape
    n_items = item_emb.shape[0]
    fn = _sc_bprmf(B, K, n_users, n_items)
    return fn(u, i_pos, i_neg, user_emb, item_emb)


# speedup vs baseline: 1.0041x; 1.0041x over previous
import functools

import jax
import jax.numpy as jnp
from jax import lax
from jax.experimental import pallas as pl
from jax.experimental.pallas import tpu as pltpu
from jax.experimental.pallas import tpu_sc as plsc

_NC = 2
_NS = 16
_L = 16


def _sc_bprmf(B, K, n_users, n_items):
    NW = _NC * _NS
    n = B // NW
    CH = 256
    NCH = n // CH
    NG = CH // _L

    mesh = plsc.VectorSubcoreMesh(core_axis_name="c", subcore_axis_name="s")

    @functools.partial(
        pl.kernel,
        mesh=mesh,
        out_type=(
            jax.ShapeDtypeStruct((B,), jnp.float32),
            jax.ShapeDtypeStruct((B,), jnp.float32),
        ),
        scratch_types=[
            pltpu.VMEM((n,), jnp.int32),
            pltpu.VMEM((n,), jnp.int32),
            pltpu.VMEM((n,), jnp.int32),
            pltpu.VMEM((CH, K), jnp.float32),
            pltpu.VMEM((CH, K), jnp.float32),
            pltpu.VMEM((CH, K), jnp.float32),
            pltpu.VMEM((n,), jnp.float32),
            pltpu.VMEM((n,), jnp.float32),
            pltpu.SemaphoreType.DMA,
        ],
        compiler_params=pltpu.CompilerParams(needs_layout_passes=False),
    )
    def sc_kernel(u_hbm, ip_hbm, in_hbm, ue_hbm, ie_hbm, sp_hbm, sn_hbm,
                  u_idx, ip_idx, in_idx, ue_v, ipv, inv, sp_v, sn_v, sem):
        wid = lax.axis_index("s") * _NC + lax.axis_index("c")
        base = wid * n

        off = pl.ds(base, n)
        pltpu.sync_copy(u_hbm.at[off], u_idx)
        pltpu.sync_copy(ip_hbm.at[off], ip_idx)
        pltpu.sync_copy(in_hbm.at[off], in_idx)

        lanes = lax.iota(jnp.int32, _L)

        def chunk_body(c, carry):
            c0 = pl.multiple_of(c * CH, CH)

            def fire_body(g, carry2):
                r0 = pl.multiple_of(g * _L, _L)
                uvec = u_idx[pl.ds(c0 + r0, _L)]
                pvec = ip_idx[pl.ds(c0 + r0, _L)]
                nvec = in_idx[pl.ds(c0 + r0, _L)]
                for t in range(_L):
                    r = pl.ds(r0 + t, 1)
                    pltpu.async_copy(
                        ue_hbm.at[pl.ds(uvec[t], 1)], ue_v.at[r], sem)
                    pltpu.async_copy(
                        ie_hbm.at[pl.ds(pvec[t], 1)], ipv.at[r], sem)
                    pltpu.async_copy(
                        ie_hbm.at[pl.ds(nvec[t], 1)], inv.at[r], sem)
                return carry2

            lax.fori_loop(0, NG, fire_body, 0)

            pltpu.make_async_copy(ue_hbm.at[pl.ds(0, CH)], ue_v, sem).wait()
            pltpu.make_async_copy(ie_hbm.at[pl.ds(0, CH)], ipv, sem).wait()
            pltpu.make_async_copy(ie_hbm.at[pl.ds(0, CH)], inv, sem).wait()

            def group_body(g, carry2):
                row0 = pl.multiple_of(g * _L, _L)
                rows = row0 + lanes
                acc_p = jnp.zeros((_L,), jnp.float32)
                acc_n = jnp.zeros((_L,), jnp.float32)
                for k in range(K):
                    col = jnp.full((_L,), k, jnp.int32)
                    ue_k = plsc.load_gather(ue_v, [rows, col])
                    ip_k = plsc.load_gather(ipv, [rows, col])
                    in_k = plsc.load_gather(inv, [rows, col])
                    acc_p = acc_p + ue_k * ip_k
                    acc_n = acc_n + ue_k * in_k
                sp_v[pl.ds(c0 + row0, _L)] = acc_p
                sn_v[pl.ds(c0 + row0, _L)] = acc_n
                return carry2

            lax.fori_loop(0, NG, group_body, 0)
            return carry

        lax.fori_loop(0, NCH, chunk_body, 0)

        out_off = pl.ds(base, n)
        pltpu.sync_copy(sp_v, sp_hbm.at[out_off])
        pltpu.sync_copy(sn_v, sn_hbm.at[out_off])

    return sc_kernel


def kernel(u, i_pos, i_neg, user_emb, item_emb):
    B = u.shape[0]
    n_users, K = user_emb.shape
    n_items = item_emb.shape[0]
    fn = _sc_bprmf(B, K, n_users, n_items)
    return fn(u, i_pos, i_neg, user_emb, item_emb)
